# baseline (speedup 1.0000x reference)
import functools

import jax
import jax.numpy as jnp
from jax import lax
from jax.experimental import pallas as pl
from jax.experimental.pallas import tpu as pltpu
from jax.experimental.pallas import tpu_sc as plsc

VOCAB = 1_000_000
DIM = 64
BATCH = 4096
SEQ = 200

_INFO = plsc.get_sparse_core_info()
_NC = _INFO.num_cores
_NS = _INFO.num_subcores
_NW = _NC * _NS

_VB = 128
_NVB = VOCAB // _VB
_A_ITERS = (_NVB + _NW - 1) // _NW
_TAIL = VOCAB - _NVB * _VB


def _relayout_kernel(tabT_hbm, tail_hbm, scr_hbm, stage, rows, gsem, osem):
    wid = lax.axis_index("s") * _NC + lax.axis_index("c")

    iota = lax.iota(jnp.int32, 16)
    dvecs = [dg * 16 + iota for dg in range(DIM // 16)]

    def fire_in(vb, buf):
        col0 = pl.multiple_of(vb * _VB, _VB)
        for td in range(DIM // 8):
            pltpu.async_copy(
                tabT_hbm.at[pl.ds(td * 8, 8), pl.ds(col0, _VB)],
                stage.at[buf, pl.ds(td * 8, 8)],
                gsem.at[buf],
            )

    def wait_in(buf):
        pltpu.make_async_copy(
            tabT_hbm.at[pl.ds(0, DIM), pl.ds(0, _VB)], stage.at[buf],
            gsem.at[buf],
        ).wait()

    def transpose(buf):
        def tbody(t2, carry):
            for tpar in range(2):
                t = t2 * 2 + tpar
                tvec = jnp.full((16,), 0, jnp.int32) + t
                for dg in range(DIM // 16):
                    v = plsc.load_gather(stage.at[buf], [dvecs[dg], tvec])
                    rows[buf, t2, pl.ds(tpar * 64 + dg * 16, 16)] = v
            return carry

        lax.fori_loop(0, _VB // 2, tbody, 0)

    def fire_out(vb, buf):
        row0 = pl.multiple_of(vb * (_VB // 2), _VB // 2)
        pltpu.async_copy(
            rows.at[buf], scr_hbm.at[pl.ds(row0, DIM)], osem.at[buf],
        )

    def wait_out(buf):
        pltpu.make_async_copy(
            scr_hbm.at[pl.ds(0, DIM)], rows.at[buf], osem.at[buf],
        ).wait()

    fire_in(wid, 0)

    def body(i, carry):
        vb = wid + _NW * i
        buf = lax.rem(i, 2)

        @pl.when(vb + _NW < _NVB)
        def _():
            fire_in(vb + _NW, lax.rem(i + 1, 2))

        @pl.when(vb < _NVB)
        def _():
            wait_in(buf)

            @pl.when(i >= 2)
            def _():
                wait_out(buf)

            transpose(buf)
            fire_out(vb, buf)

        return carry

    lax.fori_loop(0, _A_ITERS, body, 0)
    wait_out(0)
    wait_out(1)

    @pl.when(wid == 4)
    def _tail():
        pltpu.sync_copy(tail_hbm, stage.at[0])

        def tbody(t2, carry):
            for tpar in range(2):
                t = t2 * 2 + tpar
                for dg in range(DIM // 16):
                    v = stage[0, t, pl.ds(dg * 16, 16)]
                    rows[0, t2, pl.ds(tpar * 64 + dg * 16, 16)] = v
            return carry

        lax.fori_loop(0, _TAIL // 2, tbody, 0)
        pltpu.sync_copy(
            rows.at[0, pl.ds(0, _TAIL * DIM // 128)],
            scr_hbm.at[pl.ds(_NVB * (_VB // 2), _TAIL * DIM // 128)],
        )


_UT = 256
_UNITS_PER_S = BATCH // _UT
_NUNITS = SEQ * _UNITS_PER_S
_UPW = _NUNITS // _NW
_NTB = _UT // 128
_OB = 8 * _NTB * 8 * 128
_S_STRIDE = 8 * 32 * 8 * 128
_TD_STRIDE = 32 * 8 * 128


def _gather_kernel(scr_hbm, textT_hbm, out_hbm, idxb, rows, obuf,
                   gsem, osem):
    wid = lax.axis_index("s") * _NC + lax.axis_index("c")
    u0 = wid * _UPW

    iota = lax.iota(jnp.int32, 16)
    pats = [
        (2 * dg + lax.shift_right_logical(iota, 3)) * (_NTB * 1024)
        + (iota & 7) * 128
        for dg in range(DIM // 16)
    ]

    def stage_unit(u, buf):
        s = lax.div(u, _UNITS_PER_S)
        c = lax.rem(u, _UNITS_PER_S)
        pltpu.sync_copy(textT_hbm.at[s, pl.ds(c * _UT, _UT)], idxb.at[buf])
        for j in range(_UT // 128):
            pltpu.async_copy(
                scr_hbm.at[idxb.at[buf, pl.ds(j * 128, 128)]],
                rows.at[buf, pl.ds(j * 128, 128)],
                gsem.at[buf],
            )

    def wait_gathers(buf):
        pltpu.make_async_copy(
            scr_hbm.at[pl.ds(0, _UT)], rows.at[buf], gsem.at[buf],
        ).wait()

    def transpose(buf):
        def tbody(tblk, carry):
            t0 = tblk * 8
            base0 = lax.div(t0, 128) * 1024 + lax.rem(t0, 128)
            for tt in range(8):
                t = t0 + tt
                for dg in range(DIM // 16):
                    v = rows[buf, t, pl.ds(dg * 16, 16)]
                    plsc.store_scatter(
                        obuf.at[buf], [pats[dg] + (base0 + tt)], v)
            return carry

        lax.fori_loop(0, _UT // 8, tbody, 0)

    def fire_out(u, buf):
        s = lax.div(u, _UNITS_PER_S)
        c = lax.rem(u, _UNITS_PER_S)
        off = s * _S_STRIDE + c * (_NTB * 1024)
        for td in range(8):
            pltpu.async_copy(
                obuf.at[buf, pl.ds(td * (_NTB * 1024), _NTB * 1024)],
                out_hbm.at[pl.ds(
                    pl.multiple_of(off + td * _TD_STRIDE, _NTB * 1024),
                    _NTB * 1024)],
                osem.at[buf],
            )

    def wait_out(buf):
        pltpu.make_async_copy(
            out_hbm.at[pl.ds(0, _OB)], obuf.at[buf], osem.at[buf],
        ).wait()

    stage_unit(u0, 0)

    def body(i, carry):
        u = u0 + i
        buf = lax.rem(i, 2)

        @pl.when(i + 1 < _UPW)
        def _():
            stage_unit(u + 1, lax.rem(i + 1, 2))

        wait_gathers(buf)

        @pl.when(i >= 2)
        def _():
            wait_out(buf)

        transpose(buf)
        fire_out(u, buf)
        return carry

    lax.fori_loop(0, _UPW, body, 0)
    wait_out(0)
    wait_out(1)


@jax.jit
def kernel(text, img, table):
    del img
    mesh = plsc.VectorSubcoreMesh(core_axis_name="c", subcore_axis_name="s")

    tabT = table.T
    textT = text.T
    tail2d = jnp.pad(table[_NVB * _VB:], ((0, 0), (0, DIM)))

    scr = pl.kernel(
        _relayout_kernel,
        out_type=jax.ShapeDtypeStruct((VOCAB // 2, 2 * DIM), jnp.float32),
        mesh=mesh,
        scratch_types=[
            pltpu.VMEM((2, DIM, _VB), jnp.float32),
            pltpu.VMEM((2, DIM, _VB), jnp.float32),
            pltpu.SemaphoreType.DMA((2,)),
            pltpu.SemaphoreType.DMA((2,)),
        ],
        compiler_params=pltpu.CompilerParams(use_tc_tiling_on_sc=True,
                                             needs_layout_passes=False),
    )(tabT, tail2d)
    scr2d = scr.reshape(VOCAB, DIM)

    out1d = pl.kernel(
        _gather_kernel,
        out_type=jax.ShapeDtypeStruct((SEQ * DIM * BATCH,), jnp.float32),
        mesh=mesh,
        scratch_types=[
            pltpu.VMEM((2, _UT), jnp.int32),
            pltpu.VMEM((2, _UT, DIM), jnp.float32),
            pltpu.VMEM((2, _OB), jnp.float32),
            pltpu.SemaphoreType.DMA((2,)),
            pltpu.SemaphoreType.DMA((2,)),
        ],
        compiler_params=pltpu.CompilerParams(use_tc_tiling_on_sc=False,
                                             needs_layout_passes=False),
    )(scr2d, textT)

    out5 = out1d.reshape(SEQ, 8, BATCH // 128, 8, 128)
    return out5.transpose(2, 4, 0, 1, 3).reshape(BATCH, SEQ, DIM)


# speedup vs baseline: 1.5630x; 1.5630x over previous
import jax
import jax.numpy as jnp
from jax import lax
from jax.experimental import pallas as pl
from jax.experimental.pallas import tpu as pltpu
from jax.experimental.pallas import tpu_sc as plsc

VOCAB = 1_000_000
DIM = 64
BATCH = 4096
SEQ = 200

_INFO = plsc.get_sparse_core_info()
_NC = _INFO.num_cores
_NS = _INFO.num_subcores
_NW = _NC * _NS

_VB = 128
_NVB = VOCAB // _VB
_A_ITERS = (_NVB + _NW - 1) // _NW
_TAIL = VOCAB - _NVB * _VB


def _relayout_kernel(tabT_hbm, tail_hbm, scr_hbm,
                     stage0, stage1, rows0, rows1, gsem, osem):
    wid = lax.axis_index("s") * _NC + lax.axis_index("c")
    stages = (stage0, stage1)
    rowss = (rows0, rows1)

    iota = lax.iota(jnp.int32, 16)
    tokpats = [iota * DIM + tg * 16 * DIM for tg in range(_VB // 16)]

    def fire_in(vb, buf):
        col0 = pl.multiple_of(vb * _VB, _VB)
        for td in range(DIM // 8):
            pltpu.async_copy(
                tabT_hbm.at[pl.ds(td * 8, 8), pl.ds(col0, _VB)],
                stages[buf].at[pl.ds(td * 8, 8)],
                gsem.at[buf],
            )

    def wait_in(buf):
        pltpu.make_async_copy(
            tabT_hbm.at[pl.ds(0, DIM), pl.ds(0, _VB)], stages[buf],
            gsem.at[buf],
        ).wait()

    def transpose(buf):
        stage, rows = stages[buf], rowss[buf]

        @plsc.parallel_loop(0, DIM, unroll=4)
        def _(d):
            for tg in range(_VB // 16):
                v = stage[d, pl.ds(tg * 16, 16)]
                plsc.store_scatter(rows, [tokpats[tg] + d], v)

    def fire_out(vb, buf):
        off = pl.multiple_of(vb * (_VB * DIM), _VB * DIM)
        pltpu.async_copy(
            rowss[buf], scr_hbm.at[pl.ds(off, _VB * DIM)], osem.at[buf],
        )

    def wait_out(buf):
        pltpu.make_async_copy(
            scr_hbm.at[pl.ds(0, _VB * DIM)], rowss[buf], osem.at[buf],
        ).wait()

    def process(i, vb, buf):
        @pl.when(vb + _NW < _NVB)
        def _():
            fire_in(vb + _NW, 1 - buf)

        @pl.when(vb < _NVB)
        def _():
            wait_in(buf)

            @pl.when(i >= 2)
            def _():
                wait_out(buf)

            transpose(buf)
            fire_out(vb, buf)

    fire_in(wid, 0)

    def body(i2, carry):
        for sub in range(2):
            i = i2 * 2 + sub
            process(i, wid + _NW * i, sub)
        return carry

    lax.fori_loop(0, _A_ITERS // 2, body, 0)
    process(_A_ITERS - 1, wid + _NW * (_A_ITERS - 1), 0)
    wait_out(0)
    wait_out(1)

    @pl.when(wid == 4)
    def _tail():
        pltpu.sync_copy(tail_hbm, stage0)

        @plsc.parallel_loop(0, _TAIL, unroll=4)
        def _(t):
            for dg in range(DIM // 16):
                v = stage0[t, pl.ds(dg * 16, 16)]
                rows0[pl.ds(t * DIM + dg * 16, 16)] = v

        pltpu.sync_copy(
            rows0.at[pl.ds(0, _TAIL * DIM)],
            scr_hbm.at[pl.ds(_NVB * _VB * DIM, _TAIL * DIM)],
        )


_UT = 256
_UNITS_PER_S = BATCH // _UT
_NUNITS = SEQ * _UNITS_PER_S
_UPW = _NUNITS // _NW
_NTB = _UT // 128
_OB = 8 * _NTB * 8 * 128
_S_STRIDE = 8 * 32 * 8 * 128
_TD_STRIDE = 32 * 8 * 128


def _gather_kernel(scr_hbm, textT_hbm, out_hbm,
                   idxb0, idxb1, rows0, rows1, obuf0, obuf1, gsem, osem):
    wid = lax.axis_index("s") * _NC + lax.axis_index("c")
    u0 = wid * _UPW
    idxbs = (idxb0, idxb1)
    rowss = (rows0, rows1)
    obufs = (obuf0, obuf1)

    iota = lax.iota(jnp.int32, 16)
    pats = [
        (2 * dg + lax.shift_right_logical(iota, 3)) * (_NTB * 1024)
        + (iota & 7) * 128
        for dg in range(DIM // 16)
    ]

    def stage_unit(u, buf):
        s = lax.div(u, _UNITS_PER_S)
        c = lax.rem(u, _UNITS_PER_S)
        pltpu.sync_copy(textT_hbm.at[s, pl.ds(c * _UT, _UT)], idxbs[buf])
        for j in range(_UT // 128):
            pltpu.async_copy(
                scr_hbm.at[idxbs[buf].at[pl.ds(j * 128, 128)]],
                rowss[buf].at[pl.ds(j * 128, 128)],
                gsem.at[buf],
            )

    def wait_gathers(buf):
        pltpu.make_async_copy(
            scr_hbm.at[pl.ds(0, _UT)], rowss[buf], gsem.at[buf],
        ).wait()

    def transpose(buf):
        rows, obuf = rowss[buf], obufs[buf]

        @plsc.parallel_loop(0, _UT, unroll=8)
        def _(t):
            base = lax.div(t, 128) * 1024 + lax.rem(t, 128)
            for dg in range(DIM // 16):
                v = rows[t, pl.ds(dg * 16, 16)]
                plsc.store_scatter(obuf, [pats[dg] + base], v)

    def fire_out(u, buf):
        s = lax.div(u, _UNITS_PER_S)
        c = lax.rem(u, _UNITS_PER_S)
        off = s * _S_STRIDE + c * (_NTB * 1024)
        for td in range(8):
            pltpu.async_copy(
                obufs[buf].at[pl.ds(td * (_NTB * 1024), _NTB * 1024)],
                out_hbm.at[pl.ds(
                    pl.multiple_of(off + td * _TD_STRIDE, _NTB * 1024),
                    _NTB * 1024)],
                osem.at[buf],
            )

    def wait_out(buf):
        pltpu.make_async_copy(
            out_hbm.at[pl.ds(0, _OB)], obufs[buf], osem.at[buf],
        ).wait()

    def process(i, buf):
        u = u0 + i

        @pl.when(i + 1 < _UPW)
        def _():
            stage_unit(u + 1, 1 - buf)

        wait_gathers(buf)

        @pl.when(i >= 2)
        def _():
            wait_out(buf)

        transpose(buf)
        fire_out(u, buf)

    stage_unit(u0, 0)

    def body(i2, carry):
        for sub in range(2):
            process(i2 * 2 + sub, sub)
        return carry

    lax.fori_loop(0, _UPW // 2, body, 0)
    wait_out(0)
    wait_out(1)


@jax.jit
def kernel(text, img, table):
    del img
    mesh = plsc.VectorSubcoreMesh(core_axis_name="c", subcore_axis_name="s")

    tabT = table.T
    textT = text.T
    tail2d = jnp.pad(table[_NVB * _VB:], ((0, 0), (0, DIM)))

    scr = pl.kernel(
        _relayout_kernel,
        out_type=jax.ShapeDtypeStruct((VOCAB * DIM,), jnp.float32),
        mesh=mesh,
        scratch_types=[
            pltpu.VMEM((DIM, _VB), jnp.float32),
            pltpu.VMEM((DIM, _VB), jnp.float32),
            pltpu.VMEM((DIM * _VB,), jnp.float32),
            pltpu.VMEM((DIM * _VB,), jnp.float32),
            pltpu.SemaphoreType.DMA((2,)),
            pltpu.SemaphoreType.DMA((2,)),
        ],
        compiler_params=pltpu.CompilerParams(use_tc_tiling_on_sc=True,
                                             needs_layout_passes=False),
    )(tabT, tail2d)
    scr2d = scr.reshape(VOCAB, DIM)

    out1d = pl.kernel(
        _gather_kernel,
        out_type=jax.ShapeDtypeStruct((SEQ * DIM * BATCH,), jnp.float32),
        mesh=mesh,
        scratch_types=[
            pltpu.VMEM((_UT,), jnp.int32),
            pltpu.VMEM((_UT,), jnp.int32),
            pltpu.VMEM((_UT, DIM), jnp.float32),
            pltpu.VMEM((_UT, DIM), jnp.float32),
            pltpu.VMEM((_OB,), jnp.float32),
            pltpu.VMEM((_OB,), jnp.float32),
            pltpu.SemaphoreType.DMA((2,)),
            pltpu.SemaphoreType.DMA((2,)),
        ],
        compiler_params=pltpu.CompilerParams(use_tc_tiling_on_sc=False,
                                             needs_layout_passes=False),
    )(scr2d, textT)

    out5 = out1d.reshape(SEQ, 8, BATCH // 128, 8, 128)
    return out5.transpose(2, 4, 0, 1, 3).reshape(BATCH, SEQ, DIM)


# speedup vs baseline: 1.6030x; 1.0256x over previous
import jax
import jax.numpy as jnp
from jax import lax
from jax.experimental import pallas as pl
from jax.experimental.pallas import tpu as pltpu
from jax.experimental.pallas import tpu_sc as plsc

VOCAB = 1_000_000
DIM = 64
BATCH = 4096
SEQ = 200

_INFO = plsc.get_sparse_core_info()
_NC = _INFO.num_cores
_NS = _INFO.num_subcores
_NW = _NC * _NS

_VB = 256
_NVB = VOCAB // _VB
_A_ITERS = (_NVB + _NW - 1) // _NW
_TAIL = VOCAB - _NVB * _VB


def _relayout_kernel(tabT_hbm, tail_hbm, scr_hbm,
                     stage0, stage1, rows0, rows1, gsem, osem):
    wid = lax.axis_index("s") * _NC + lax.axis_index("c")
    stages = (stage0, stage1)
    rowss = (rows0, rows1)

    iota = lax.iota(jnp.int32, 16)
    tokpats = [iota * DIM + tg * 16 * DIM for tg in range(_VB // 16)]

    def fire_in(vb, buf):
        col0 = pl.multiple_of(vb * _VB, _VB)
        pltpu.async_copy(
            tabT_hbm.at[pl.ds(0, DIM), pl.ds(col0, _VB)],
            stages[buf],
            gsem.at[buf],
        )

    def wait_in(buf):
        pltpu.make_async_copy(
            tabT_hbm.at[pl.ds(0, DIM), pl.ds(0, _VB)], stages[buf],
            gsem.at[buf],
        ).wait()

    def transpose(buf):
        stage, rows = stages[buf], rowss[buf]

        @plsc.parallel_loop(0, DIM, unroll=4)
        def _(d):
            for tg in range(_VB // 16):
                v = stage[d, pl.ds(tg * 16, 16)]
                plsc.store_scatter(rows, [tokpats[tg] + d], v)

    def fire_out(vb, buf):
        off = pl.multiple_of(vb * (_VB * DIM), _VB * DIM)
        pltpu.async_copy(
            rowss[buf], scr_hbm.at[pl.ds(off, _VB * DIM)], osem.at[buf],
        )

    def wait_out(buf):
        pltpu.make_async_copy(
            scr_hbm.at[pl.ds(0, _VB * DIM)], rowss[buf], osem.at[buf],
        ).wait()

    def process(i, vb, buf):
        @pl.when(vb + _NW < _NVB)
        def _():
            fire_in(vb + _NW, 1 - buf)

        @pl.when(vb < _NVB)
        def _():
            wait_in(buf)

            @pl.when(i >= 2)
            def _():
                wait_out(buf)

            transpose(buf)
            fire_out(vb, buf)

    fire_in(wid, 0)

    def body(i2, carry):
        for sub in range(2):
            i = i2 * 2 + sub
            process(i, wid + _NW * i, sub)
        return carry

    lax.fori_loop(0, _A_ITERS // 2, body, 0)
    process(_A_ITERS - 1, wid + _NW * (_A_ITERS - 1), 0)
    wait_out(0)
    wait_out(1)

    @pl.when(wid == 4)
    def _tail():
        pltpu.sync_copy(tail_hbm, stage0.at[pl.ds(0, DIM), pl.ds(0, 128)])

        @plsc.parallel_loop(0, _TAIL, unroll=4)
        def _(t):
            for dg in range(DIM // 16):
                v = stage0[t, pl.ds(dg * 16, 16)]
                rows0[pl.ds(t * DIM + dg * 16, 16)] = v

        pltpu.sync_copy(
            rows0.at[pl.ds(0, _TAIL * DIM)],
            scr_hbm.at[pl.ds(_NVB * _VB * DIM, _TAIL * DIM)],
        )


_UT = 256
_UNITS_PER_S = BATCH // _UT
_NUNITS = SEQ * _UNITS_PER_S
_UPW = _NUNITS // _NW
_NTB = _UT // 128
_OB = 8 * _NTB * 8 * 128
_S_STRIDE = 8 * 32 * 8 * 128
_TD_STRIDE = 32 * 8 * 128


_TPW = _UPW * _UT


def _gather_kernel(scr_hbm, textF_hbm, out_hbm,
                   idxfull, rows0, rows1, obuf0, obuf1, gsem, osem):
    wid = lax.axis_index("s") * _NC + lax.axis_index("c")
    u0 = wid * _UPW
    rowss = (rows0, rows1)
    obufs = (obuf0, obuf1)

    iota = lax.iota(jnp.int32, 16)
    pats = [
        (2 * dg + lax.shift_right_logical(iota, 3)) * (_NTB * 1024)
        + (iota & 7) * 128
        for dg in range(DIM // 16)
    ]

    def fire_gathers(i, buf):
        for j in range(_UT // 128):
            off = pl.multiple_of(i * _UT + j * 128, 128)
            pltpu.async_copy(
                scr_hbm.at[idxfull.at[pl.ds(off, 128)]],
                rowss[buf].at[pl.ds(j * 128, 128)],
                gsem.at[buf],
            )

    def wait_gathers(buf):
        pltpu.make_async_copy(
            scr_hbm.at[pl.ds(0, _UT)], rowss[buf], gsem.at[buf],
        ).wait()

    def transpose(buf):
        rows, obuf = rowss[buf], obufs[buf]

        @plsc.parallel_loop(0, _UT, unroll=8)
        def _(t):
            base = lax.div(t, 128) * 1024 + lax.rem(t, 128)
            for dg in range(DIM // 16):
                v = rows[t, pl.ds(dg * 16, 16)]
                plsc.store_scatter(obuf, [pats[dg] + base], v)

    def fire_out(u, buf):
        s = lax.div(u, _UNITS_PER_S)
        c = lax.rem(u, _UNITS_PER_S)
        off = s * _S_STRIDE + c * (_NTB * 1024)
        for td in range(8):
            pltpu.async_copy(
                obufs[buf].at[pl.ds(td * (_NTB * 1024), _NTB * 1024)],
                out_hbm.at[pl.ds(
                    pl.multiple_of(off + td * _TD_STRIDE, _NTB * 1024),
                    _NTB * 1024)],
                osem.at[buf],
            )

    def wait_out(buf):
        pltpu.make_async_copy(
            out_hbm.at[pl.ds(0, _OB)], obufs[buf], osem.at[buf],
        ).wait()

    def process(i, buf):
        u = u0 + i

        @pl.when(i + 1 < _UPW)
        def _():
            fire_gathers(i + 1, 1 - buf)

        wait_gathers(buf)

        @pl.when(i >= 2)
        def _():
            wait_out(buf)

        transpose(buf)
        fire_out(u, buf)

    pltpu.sync_copy(
        textF_hbm.at[pl.ds(pl.multiple_of(wid * _TPW, 1024), _TPW)], idxfull)
    fire_gathers(0, 0)

    def body(i2, carry):
        for sub in range(2):
            process(i2 * 2 + sub, sub)
        return carry

    lax.fori_loop(0, _UPW // 2, body, 0)
    wait_out(0)
    wait_out(1)


@jax.jit
def kernel(text, img, table):
    del img
    mesh = plsc.VectorSubcoreMesh(core_axis_name="c", subcore_axis_name="s")

    tabT = table.T
    textF = text.T.reshape(-1)
    tail2d = jnp.pad(table[_NVB * _VB:], ((0, 0), (0, DIM)))

    scr = pl.kernel(
        _relayout_kernel,
        out_type=jax.ShapeDtypeStruct((VOCAB * DIM,), jnp.float32),
        mesh=mesh,
        scratch_types=[
            pltpu.VMEM((DIM, _VB), jnp.float32),
            pltpu.VMEM((DIM, _VB), jnp.float32),
            pltpu.VMEM((DIM * _VB,), jnp.float32),
            pltpu.VMEM((DIM * _VB,), jnp.float32),
            pltpu.SemaphoreType.DMA((2,)),
            pltpu.SemaphoreType.DMA((2,)),
        ],
        compiler_params=pltpu.CompilerParams(use_tc_tiling_on_sc=True,
                                             needs_layout_passes=False),
    )(tabT, tail2d)
    scr2d = scr.reshape(VOCAB, DIM)

    out1d = pl.kernel(
        _gather_kernel,
        out_type=jax.ShapeDtypeStruct((SEQ * DIM * BATCH,), jnp.float32),
        mesh=mesh,
        scratch_types=[
            pltpu.VMEM((_TPW,), jnp.int32),
            pltpu.VMEM((_UT, DIM), jnp.float32),
            pltpu.VMEM((_UT, DIM), jnp.float32),
            pltpu.VMEM((_OB,), jnp.float32),
            pltpu.VMEM((_OB,), jnp.float32),
            pltpu.SemaphoreType.DMA((2,)),
            pltpu.SemaphoreType.DMA((2,)),
        ],
        compiler_params=pltpu.CompilerParams(use_tc_tiling_on_sc=False,
                                             needs_layout_passes=False),
    )(scr2d, textF)

    out5 = out1d.reshape(SEQ, 8, BATCH // 128, 8, 128)
    return out5.transpose(2, 4, 0, 1, 3).reshape(BATCH, SEQ, DIM)


# speedup vs baseline: 1.9478x; 1.2151x over previous
import jax
import jax.numpy as jnp
from jax import lax
from jax.experimental import pallas as pl
from jax.experimental.pallas import tpu as pltpu
from jax.experimental.pallas import tpu_sc as plsc

VOCAB = 1_000_000
DIM = 64
BATCH = 4096
SEQ = 200

_INFO = plsc.get_sparse_core_info()
_NC = _INFO.num_cores
_NS = _INFO.num_subcores
_NW = _NC * _NS

_VB = 256
_NVB = VOCAB // _VB
_A_ITERS = (_NVB + _NW - 1) // _NW
_TAIL = VOCAB - _NVB * _VB


def _relayout_kernel(tabT_hbm, tail_hbm, scr_hbm,
                     stage0, stage1, rows0, rows1, gsem, osem):
    wid = lax.axis_index("s") * _NC + lax.axis_index("c")
    stages = (stage0, stage1)
    rowss = (rows0, rows1)

    iota = lax.iota(jnp.int32, 16)
    tokpats = [iota * DIM + tg * 16 * DIM for tg in range(_VB // 16)]

    def fire_in(vb, buf):
        col0 = pl.multiple_of(vb * _VB, _VB)
        pltpu.async_copy(
            tabT_hbm.at[pl.ds(0, DIM), pl.ds(col0, _VB)],
            stages[buf],
            gsem.at[buf],
        )

    def wait_in(buf):
        pltpu.make_async_copy(
            tabT_hbm.at[pl.ds(0, DIM), pl.ds(0, _VB)], stages[buf],
            gsem.at[buf],
        ).wait()

    def transpose(buf):
        stage, rows = stages[buf], rowss[buf]

        @plsc.parallel_loop(0, DIM, unroll=4)
        def _(d):
            for tg in range(_VB // 16):
                v = stage[d, pl.ds(tg * 16, 16)]
                plsc.store_scatter(rows, [tokpats[tg] + d], v)

    def fire_out(vb, buf):
        off = pl.multiple_of(vb * (_VB * DIM), _VB * DIM)
        pltpu.async_copy(
            rowss[buf], scr_hbm.at[pl.ds(off, _VB * DIM)], osem.at[buf],
        )

    def wait_out(buf):
        pltpu.make_async_copy(
            scr_hbm.at[pl.ds(0, _VB * DIM)], rowss[buf], osem.at[buf],
        ).wait()

    def process(i, vb, buf):
        @pl.when(vb + _NW < _NVB)
        def _():
            fire_in(vb + _NW, 1 - buf)

        @pl.when(vb < _NVB)
        def _():
            wait_in(buf)

            @pl.when(i >= 2)
            def _():
                wait_out(buf)

            transpose(buf)
            fire_out(vb, buf)

    fire_in(wid, 0)

    def body(i2, carry):
        for sub in range(2):
            i = i2 * 2 + sub
            process(i, wid + _NW * i, sub)
        return carry

    lax.fori_loop(0, _A_ITERS // 2, body, 0)
    process(_A_ITERS - 1, wid + _NW * (_A_ITERS - 1), 0)
    wait_out(0)
    wait_out(1)

    @pl.when(wid == 4)
    def _tail():
        pltpu.sync_copy(tail_hbm, stage0.at[pl.ds(0, DIM), pl.ds(0, 128)])

        @plsc.parallel_loop(0, _TAIL, unroll=4)
        def _(t):
            for dg in range(DIM // 16):
                v = stage0[t, pl.ds(dg * 16, 16)]
                rows0[pl.ds(t * DIM + dg * 16, 16)] = v

        pltpu.sync_copy(
            rows0.at[pl.ds(0, _TAIL * DIM)],
            scr_hbm.at[pl.ds(_NVB * _VB * DIM, _TAIL * DIM)],
        )


_UT = 256
_UNITS_PER_S = BATCH // _UT
_NUNITS = SEQ * _UNITS_PER_S
_UPW = _NUNITS // _NW
_NTB = _UT // 128
_OB = 8 * _NTB * 8 * 128
_S_STRIDE = 8 * 32 * 8 * 128
_TD_STRIDE = 32 * 8 * 128


_TPW = _UPW * _UT


def _gather_kernel(scr_hbm, textF_hbm, out_hbm,
                   idxfull, rows0, rows1, obuf0, obuf1, gsem, osem):
    wid = lax.axis_index("s") * _NC + lax.axis_index("c")
    u0 = wid * _UPW
    rowss = (rows0, rows1)
    obufs = (obuf0, obuf1)

    iota = lax.iota(jnp.int32, 16)
    pats = [
        (2 * dg + lax.shift_right_logical(iota, 3)) * (_NTB * 1024)
        + (iota & 7) * 128
        for dg in range(DIM // 16)
    ]

    def fire_gathers(i, buf):
        for j in range(_UT // 128):
            off = pl.multiple_of(i * _UT + j * 128, 128)
            pltpu.async_copy(
                scr_hbm.at[idxfull.at[pl.ds(off, 128)]],
                rowss[buf].at[pl.ds(j * 128, 128)],
                gsem.at[buf],
            )

    def wait_gathers(buf):
        pltpu.make_async_copy(
            scr_hbm.at[pl.ds(0, _UT)], rowss[buf], gsem.at[buf],
        ).wait()

    def transpose(buf):
        rows, obuf = rowss[buf], obufs[buf]

        @plsc.parallel_loop(0, _UT, unroll=8)
        def _(t):
            base = lax.div(t, 128) * 1024 + lax.rem(t, 128)
            for dg in range(DIM // 16):
                v = rows[t, pl.ds(dg * 16, 16)]
                plsc.store_scatter(obuf, [pats[dg] + base], v)

    def fire_out(u, buf):
        s = lax.div(u, _UNITS_PER_S)
        c = lax.rem(u, _UNITS_PER_S)
        off = s * _S_STRIDE + c * (_NTB * 1024)
        for td in range(8):
            pltpu.async_copy(
                obufs[buf].at[pl.ds(td * (_NTB * 1024), _NTB * 1024)],
                out_hbm.at[pl.ds(
                    pl.multiple_of(off + td * _TD_STRIDE, _NTB * 1024),
                    _NTB * 1024)],
                osem.at[buf],
            )

    def wait_out(buf):
        pltpu.make_async_copy(
            out_hbm.at[pl.ds(0, _OB)], obufs[buf], osem.at[buf],
        ).wait()

    def process(i, buf):
        u = u0 + i

        @pl.when(i + 1 < _UPW)
        def _():
            fire_gathers(i + 1, 1 - buf)

        wait_gathers(buf)

        @pl.when(i >= 2)
        def _():
            wait_out(buf)

        transpose(buf)
        fire_out(u, buf)

    pltpu.sync_copy(
        textF_hbm.at[pl.ds(pl.multiple_of(wid * _TPW, 1024), _TPW)], idxfull)
    fire_gathers(0, 0)

    def body(i2, carry):
        for sub in range(2):
            process(i2 * 2 + sub, sub)
        return carry

    lax.fori_loop(0, _UPW // 2, body, 0)
    wait_out(0)
    wait_out(1)


@jax.jit
def kernel(text, img, table):
    del img
    mesh = plsc.VectorSubcoreMesh(core_axis_name="c", subcore_axis_name="s")

    textF = text.T.reshape(-1)
    scr2d = table

    out1d = pl.kernel(
        _gather_kernel,
        out_type=jax.ShapeDtypeStruct((SEQ * DIM * BATCH,), jnp.float32),
        mesh=mesh,
        scratch_types=[
            pltpu.VMEM((_TPW,), jnp.int32),
            pltpu.VMEM((_UT, DIM), jnp.float32),
            pltpu.VMEM((_UT, DIM), jnp.float32),
            pltpu.VMEM((_OB,), jnp.float32),
            pltpu.VMEM((_OB,), jnp.float32),
            pltpu.SemaphoreType.DMA((2,)),
            pltpu.SemaphoreType.DMA((2,)),
        ],
        compiler_params=pltpu.CompilerParams(use_tc_tiling_on_sc=False,
                                             needs_layout_passes=False),
    )(scr2d, textF)

    out5 = out1d.reshape(SEQ, 8, BATCH // 128, 8, 128)
    return out5.transpose(2, 4, 0, 1, 3).reshape(BATCH, SEQ, DIM)
